# baseline (speedup 1.0000x reference)
import functools

import jax
import jax.numpy as jnp
from jax import lax
from jax.experimental import pallas as pl
from jax.experimental.pallas import tpu as pltpu
from jax.experimental.pallas import tpu_sc as plsc

_N = 128
_DX = 0.05
_WEIGHT = 10.0
_INV = 1.0 / _DX
_NB = 4
_CH_ROWS = _N * _N * _N // 16
_LOFF = (-1, 1, -_N * _N, _N * _N, -_N, _N)
_CAP = 3072
_DCH = 16
_NDMA = _CAP // _DCH


def _make_sc_call(n_pts, nw, p_per_w):
    chunks = p_per_w // 16
    mesh = plsc.VectorSubcoreMesh(core_axis_name="c", subcore_axis_name="s")

    @functools.partial(
        pl.kernel,
        out_type=jax.ShapeDtypeStruct((nw, 16), jnp.float32),
        mesh=mesh,
        compiler_params=pltpu.CompilerParams(needs_layout_passes=False,
                                             use_tc_tiling_on_sc=False),
        scratch_types=[
            pltpu.VMEM((p_per_w,), jnp.int32),
            pltpu.VMEM((p_per_w,), jnp.int32),
            pltpu.VMEM((p_per_w,), jnp.int32),
            pltpu.VMEM((p_per_w,), jnp.float32),
            pltpu.VMEM((p_per_w,), jnp.float32),
            pltpu.VMEM((p_per_w,), jnp.float32),
            pltpu.VMEM((16 + p_per_w,), jnp.int32),
            pltpu.VMEM((6 * p_per_w,), jnp.int32),
            pltpu.VMEM((_CAP + 16,), jnp.int32),
            pltpu.VMEM((_CAP,), jnp.int32),
            pltpu.VMEM((_CAP, 16), jnp.float32),
            pltpu.VMEM((_CAP, 16), jnp.float32),
            pltpu.VMEM((16,), jnp.float32),
            pltpu.SemaphoreType.DMA,
            pltpu.SemaphoreType.DMA,
        ],
    )
    def sc_call(a_hbm, b_hbm, pts_hbm, nrm_hbm, out_hbm, xv, yv, zv,
                nxv, nyv, nzv, linv, posv, lidv, cidv, gin, gout, accv,
                sem_a, sem_b):
        wid = lax.axis_index("s") * 2 + lax.axis_index("c")
        base = wid * p_per_w
        kpad = nw * p_per_w

        for r, dst in enumerate((xv, yv, zv)):
            pltpu.sync_copy(pts_hbm.at[pl.ds(r * kpad + base, p_per_w)], dst)
        for r, dst in enumerate((nxv, nyv, nzv)):
            pltpu.sync_copy(nrm_hbm.at[pl.ds(r * kpad + base, p_per_w)], dst)

        zeros16 = jnp.zeros((16,), jnp.int32)

        def memset(j, carry):
            lidv[pl.ds(j * 16, 16)] = zeros16
            return carry

        lax.fori_loop(0, (_CAP + 16) // 16, memset, 0)

        linv[pl.ds(0, 16)] = jnp.full((16,), -(1 << 20), jnp.int32)
        first_lane = lax.iota(jnp.int32, 16) == 0

        def build(i, wptr):
            s = i * 16
            lin = (xv[pl.ds(s, 16)] * (_N * _N)
                   + yv[pl.ds(s, 16)] * _N
                   + zv[pl.ds(s, 16)])
            linv[pl.ds(16 + s, 16)] = lin
            prev = linv[pl.ds(15 + s, 16)]
            for si, off in enumerate(_LOFF):
                line = (lin + off) >> 4
                pline = (prev + off) >> 4
                isnew = (line != pline) | first_lane
                newi = isnew.astype(jnp.int32)
                pos = wptr + jnp.cumsum(newi) - 1
                posv[pl.ds(si * p_per_w + s, 16)] = pos
                plsc.store_compressed(lidv.at[pl.ds(wptr, 16)], line,
                                      mask=isnew)
                wptr = wptr + jnp.sum(newi)
            return wptr

        wptr = lax.fori_loop(0, chunks, build, jnp.int32(0))

        lane = lax.iota(jnp.int32, 16)
        acc = jnp.zeros((16,), jnp.float32)

        ngrp = (wptr + 255) // 256
        ndma = ngrp * (256 // _DCH)

        def round_body(c, acc_in):
            coff = c * _CH_ROWS

            def fire(j, carry):
                s = j * _DCH
                cidv[pl.ds(s, _DCH)] = lidv[pl.ds(s, _DCH)] + coff
                pltpu.async_copy(
                    a_hbm.at[cidv.at[pl.ds(s, _DCH)]],
                    gin.at[pl.ds(s, _DCH)], sem_a)
                pltpu.async_copy(
                    b_hbm.at[cidv.at[pl.ds(s, _DCH)]],
                    gout.at[pl.ds(s, _DCH)], sem_b)
                return carry

            def drain(j, carry):
                pltpu.make_async_copy(
                    a_hbm.at[pl.ds(0, 256)],
                    gin.at[pl.ds(j * 256, 256)], sem_a).wait()
                pltpu.make_async_copy(
                    b_hbm.at[pl.ds(0, 256)],
                    gout.at[pl.ds(j * 256, 256)], sem_b).wait()
                return carry

            lax.fori_loop(0, ndma, fire, 0)
            lax.fori_loop(0, ngrp, drain, 0)

            def extract(i, acc_c):
                s = i * 16
                lin = linv[pl.ds(16 + s, 16)]
                w0 = lin & 15
                wm = (lin - 1) & 15
                wp = (lin + 1) & 15
                pz0 = posv[pl.ds(0 * p_per_w + s, 16)]
                pz1 = posv[pl.ds(1 * p_per_w + s, 16)]
                px0 = posv[pl.ds(2 * p_per_w + s, 16)]
                px1 = posv[pl.ds(3 * p_per_w + s, 16)]
                py0 = posv[pl.ds(4 * p_per_w + s, 16)]
                py1 = posv[pl.ds(5 * p_per_w + s, 16)]
                pc = jnp.where(w0 != 0, pz0, pz1)
                ci = plsc.load_gather(gin, [pc, w0])
                ki = plsc.load_gather(gin, [pz0, wm])
                fi = plsc.load_gather(gin, [pz1, wp])
                li = plsc.load_gather(gin, [px0, w0])
                ri = plsc.load_gather(gin, [px1, w0])
                bi = plsc.load_gather(gin, [py0, w0])
                ai = plsc.load_gather(gin, [py1, w0])
                co = plsc.load_gather(gout, [pc, w0])
                ko = plsc.load_gather(gout, [pz0, wm])
                fo = plsc.load_gather(gout, [pz1, wp])
                lo = plsc.load_gather(gout, [px0, w0])
                ro = plsc.load_gather(gout, [px1, w0])
                bo = plsc.load_gather(gout, [py0, w0])
                ao = plsc.load_gather(gout, [py1, w0])
                nx = nxv[pl.ds(s, 16)]
                ny = nyv[pl.ds(s, 16)]
                nz = nzv[pl.ds(s, 16)]
                px = nx > 0
                py = ny > 0
                pz = nz > 0
                nzneg = nz < 0
                d0 = ci - co
                gxi = jnp.where(px, ci - li, ri - ci)
                gxo = jnp.where(px, ro - co, co - lo)
                gyi = jnp.where(py, ci - bi, ai - ci)
                gyo = jnp.where(py, ao - co, co - bo)
                gzi = jnp.where(pz, fi - ci, ci - ki)
                gzo = jnp.where(nzneg, fo - co, co - ko)
                ndi = gxi * nx + gyi * ny + gzi * nz
                ndo = gxo * nx + gyo * ny + gzo * nz
                dn = (ndi - ndo) * _INV
                total = d0 * d0 + dn * dn
                g = base + s + lane
                return acc_c + jnp.where(g < n_pts, total, 0.0)

            return lax.fori_loop(0, chunks, extract, acc_in)

        acc = lax.fori_loop(0, _NB, round_body, acc)

        accv[...] = acc
        pltpu.sync_copy(accv, out_hbm.at[wid])

    return sc_call


def kernel(subdomain_in, subdomain_out, x_idx, y_idx, z_idx,
           normal_x, normal_y, normal_z):
    k = x_idx.shape[0]
    nw = 32
    p_per_w = -(-k // (nw * 16)) * 16
    kpad = nw * p_per_w
    pad = kpad - k

    a = subdomain_in.reshape(_NB * _CH_ROWS, 16)
    b = subdomain_out.reshape(_NB * _CH_ROWS, 16)
    pts = jnp.pad(jnp.stack([x_idx, y_idx, z_idx]), ((0, 0), (0, pad)),
                  constant_values=64).reshape(-1)
    nrm = jnp.pad(jnp.stack([normal_x, normal_y, normal_z]),
                  ((0, 0), (0, pad))).reshape(-1)

    partials = _make_sc_call(k, nw, p_per_w)(a, b, pts, nrm)
    return jnp.sum(partials) * (_WEIGHT / (_NB * k))


# speedup vs baseline: 1.7640x; 1.7640x over previous
import functools

import jax
import jax.numpy as jnp
from jax import lax
from jax.experimental import pallas as pl
from jax.experimental.pallas import tpu as pltpu
from jax.experimental.pallas import tpu_sc as plsc

_N = 128
_DX = 0.05
_WEIGHT = 10.0
_INV = 1.0 / _DX
_NB = 4
_CH_ROWS = _N * _N * _N // 16
_LOFF = (-1, 1, -_N * _N, _N * _N, -_N, _N)
_CAP = 3072
_DCH = 16
_NDMA = _CAP // _DCH


def _make_sc_call(n_pts, nw, p_per_w):
    chunks = p_per_w // 16
    mesh = plsc.VectorSubcoreMesh(core_axis_name="c", subcore_axis_name="s")

    @functools.partial(
        pl.kernel,
        out_type=jax.ShapeDtypeStruct((nw, 16), jnp.float32),
        mesh=mesh,
        compiler_params=pltpu.CompilerParams(needs_layout_passes=False,
                                             use_tc_tiling_on_sc=False),
        scratch_types=[
            pltpu.VMEM((p_per_w,), jnp.int32),
            pltpu.VMEM((p_per_w,), jnp.int32),
            pltpu.VMEM((p_per_w,), jnp.int32),
            pltpu.VMEM((p_per_w,), jnp.float32),
            pltpu.VMEM((p_per_w,), jnp.float32),
            pltpu.VMEM((p_per_w,), jnp.float32),
            pltpu.VMEM((16 + p_per_w,), jnp.int32),
            pltpu.VMEM((6 * p_per_w,), jnp.int32),
            pltpu.VMEM((_CAP + 16,), jnp.int32),
            pltpu.VMEM((_CAP,), jnp.int32),
            pltpu.VMEM((_CAP, 16), jnp.float32),
            pltpu.VMEM((_CAP, 16), jnp.float32),
            pltpu.VMEM((16,), jnp.float32),
            pltpu.SemaphoreType.DMA,
            pltpu.SemaphoreType.DMA,
        ],
    )
    def sc_call(a_hbm, b_hbm, pts_hbm, nrm_hbm, out_hbm, xv, yv, zv,
                nxv, nyv, nzv, linv, posv, lidv, cidv, gin, gout, accv,
                sem_a, sem_b):
        wid = lax.axis_index("s") * 2 + lax.axis_index("c")
        base = wid * p_per_w
        kpad = nw * p_per_w

        for r, dst in enumerate((xv, yv, zv)):
            pltpu.sync_copy(pts_hbm.at[pl.ds(r * kpad + base, p_per_w)], dst)
        for r, dst in enumerate((nxv, nyv, nzv)):
            pltpu.sync_copy(nrm_hbm.at[pl.ds(r * kpad + base, p_per_w)], dst)

        zeros16 = jnp.zeros((16,), jnp.int32)

        def memset(j, carry):
            lidv[pl.ds(j * 16, 16)] = zeros16
            return carry

        lax.fori_loop(0, (_CAP + 16) // 16, memset, 0)

        linv[pl.ds(0, 16)] = jnp.full((16,), -(1 << 20), jnp.int32)
        first_lane = lax.iota(jnp.int32, 16) == 0

        def build(i, wptr):
            s = i * 16
            lin = (xv[pl.ds(s, 16)] * (_N * _N)
                   + yv[pl.ds(s, 16)] * _N
                   + zv[pl.ds(s, 16)])
            linv[pl.ds(16 + s, 16)] = lin
            prev = linv[pl.ds(15 + s, 16)]
            for si, off in enumerate(_LOFF):
                line = (lin + off) >> 4
                pline = (prev + off) >> 4
                isnew = (line != pline) | first_lane
                newi = isnew.astype(jnp.int32)
                pos = wptr + jnp.cumsum(newi) - 1
                posv[pl.ds(si * p_per_w + s, 16)] = pos
                plsc.store_compressed(lidv.at[pl.ds(wptr, 16)], line,
                                      mask=isnew)
                wptr = wptr + jnp.sum(newi)
            return wptr

        wptr = lax.fori_loop(0, chunks, build, jnp.int32(0))

        lane = lax.iota(jnp.int32, 16)
        acc = jnp.zeros((16,), jnp.float32)

        ndma = (wptr + (_DCH - 1)) // _DCH

        def round_body(c, acc_in):
            coff = c * _CH_ROWS

            def fire(j, carry):
                s = j * _DCH
                cidv[pl.ds(s, _DCH)] = lidv[pl.ds(s, _DCH)] + coff
                pltpu.async_copy(
                    a_hbm.at[cidv.at[pl.ds(s, _DCH)]],
                    gin.at[pl.ds(s, _DCH)], sem_a)
                pltpu.async_copy(
                    b_hbm.at[cidv.at[pl.ds(s, _DCH)]],
                    gout.at[pl.ds(s, _DCH)], sem_b)
                return carry

            def drain(j, carry):
                pltpu.make_async_copy(
                    a_hbm.at[pl.ds(0, _DCH)],
                    gin.at[pl.ds(j * _DCH, _DCH)], sem_a).wait()
                pltpu.make_async_copy(
                    b_hbm.at[pl.ds(0, _DCH)],
                    gout.at[pl.ds(j * _DCH, _DCH)], sem_b).wait()
                return carry

            lax.fori_loop(0, ndma, fire, 0)
            lax.fori_loop(0, ndma, drain, 0)

            def extract(i, acc_c):
                s = i * 16
                lin = linv[pl.ds(16 + s, 16)]
                w0 = lin & 15
                wm = (lin - 1) & 15
                wp = (lin + 1) & 15
                pz0 = posv[pl.ds(0 * p_per_w + s, 16)]
                pz1 = posv[pl.ds(1 * p_per_w + s, 16)]
                px0 = posv[pl.ds(2 * p_per_w + s, 16)]
                px1 = posv[pl.ds(3 * p_per_w + s, 16)]
                py0 = posv[pl.ds(4 * p_per_w + s, 16)]
                py1 = posv[pl.ds(5 * p_per_w + s, 16)]
                pc = jnp.where(w0 != 0, pz0, pz1)
                ci = plsc.load_gather(gin, [pc, w0])
                ki = plsc.load_gather(gin, [pz0, wm])
                fi = plsc.load_gather(gin, [pz1, wp])
                li = plsc.load_gather(gin, [px0, w0])
                ri = plsc.load_gather(gin, [px1, w0])
                bi = plsc.load_gather(gin, [py0, w0])
                ai = plsc.load_gather(gin, [py1, w0])
                co = plsc.load_gather(gout, [pc, w0])
                ko = plsc.load_gather(gout, [pz0, wm])
                fo = plsc.load_gather(gout, [pz1, wp])
                lo = plsc.load_gather(gout, [px0, w0])
                ro = plsc.load_gather(gout, [px1, w0])
                bo = plsc.load_gather(gout, [py0, w0])
                ao = plsc.load_gather(gout, [py1, w0])
                nx = nxv[pl.ds(s, 16)]
                ny = nyv[pl.ds(s, 16)]
                nz = nzv[pl.ds(s, 16)]
                px = nx > 0
                py = ny > 0
                pz = nz > 0
                nzneg = nz < 0
                d0 = ci - co
                gxi = jnp.where(px, ci - li, ri - ci)
                gxo = jnp.where(px, ro - co, co - lo)
                gyi = jnp.where(py, ci - bi, ai - ci)
                gyo = jnp.where(py, ao - co, co - bo)
                gzi = jnp.where(pz, fi - ci, ci - ki)
                gzo = jnp.where(nzneg, fo - co, co - ko)
                ndi = gxi * nx + gyi * ny + gzi * nz
                ndo = gxo * nx + gyo * ny + gzo * nz
                dn = (ndi - ndo) * _INV
                total = d0 * d0 + dn * dn
                g = base + s + lane
                return acc_c + jnp.where(g < n_pts, total, 0.0)

            return lax.fori_loop(0, chunks, extract, acc_in)

        acc = lax.fori_loop(0, _NB, round_body, acc)

        accv[...] = acc
        pltpu.sync_copy(accv, out_hbm.at[wid])

    return sc_call


def kernel(subdomain_in, subdomain_out, x_idx, y_idx, z_idx,
           normal_x, normal_y, normal_z):
    k = x_idx.shape[0]
    nw = 32
    p_per_w = -(-k // (nw * 16)) * 16
    kpad = nw * p_per_w
    pad = kpad - k

    a = subdomain_in.reshape(_NB * _CH_ROWS, 16)
    b = subdomain_out.reshape(_NB * _CH_ROWS, 16)
    pts = jnp.pad(jnp.stack([x_idx, y_idx, z_idx]), ((0, 0), (0, pad)),
                  constant_values=64).reshape(-1)
    nrm = jnp.pad(jnp.stack([normal_x, normal_y, normal_z]),
                  ((0, 0), (0, pad))).reshape(-1)

    partials = _make_sc_call(k, nw, p_per_w)(a, b, pts, nrm)
    return jnp.sum(partials) * (_WEIGHT / (_NB * k))
